# initial kernel scaffold (unmeasured)
import jax
import jax.numpy as jnp
from jax import lax
from jax.experimental import pallas as pl
from jax.experimental.pallas import tpu as pltpu

N_DEV = 4
B, Sq, Skv, Dm = 4, 256, 4096, 1024
H_LOC, Dh = 8, 128
QB = 4
NKB = 16
BLK = 64
SCALE = 0.08838834764831843


def kernel(x, Wq, K_ext, V_ext, Wo):
    K_r = K_ext.reshape(B, NKB, QB, BLK, 32, Dh)
    V_r = V_ext.reshape(B, NKB, QB, BLK, 32, Dh)

    def body(x_ref, wq_ref, k_ref, v_ref, wo_ref, out_ref,
             xall, q_all, kbuf, vbuf, ctx, po, po_rx,
             ag_send, ag_recv, rs_send, rs_recv, dma_sems):
        j = lax.axis_index("i")
        left = lax.rem(j - 1 + N_DEV, N_DEV)
        right = lax.rem(j + 1, N_DEV)

        barrier = pltpu.get_barrier_semaphore()
        for nbr in (left, right):
            pl.semaphore_signal(barrier, inc=1, device_id=(nbr,),
                                device_id_type=pl.DeviceIdType.MESH)
        pl.semaphore_wait(barrier, 2)

        xall[pl.ds(j * Sq, Sq), :] = x_ref[0]

        for h in range(N_DEV - 1):
            c = lax.rem(j - h + 2 * N_DEV, N_DEV)
            rdma = pltpu.make_async_remote_copy(
                src_ref=xall.at[pl.ds(c * Sq, Sq), :],
                dst_ref=xall.at[pl.ds(c * Sq, Sq), :],
                send_sem=ag_send.at[h],
                recv_sem=ag_recv.at[h],
                device_id=(right,),
                device_id_type=pl.DeviceIdType.MESH,
            )
            rdma.start()
            rdma.wait()

        q_all[:, :] = jnp.dot(xall[:, :], wq_ref[:, :],
                              preferred_element_type=jnp.float32)

        def attn_step(idx, carry):
            b = idx // QB
            qb = lax.rem(idx, QB)
            copies = []
            for h in range(H_LOC):
                ha = j * H_LOC + h
                ck = pltpu.make_async_copy(
                    k_ref.at[b, :, qb, :, ha, :], kbuf.at[h],
                    dma_sems.at[h])
                cv = pltpu.make_async_copy(
                    v_ref.at[b, :, qb, :, ha, :], vbuf.at[h],
                    dma_sems.at[H_LOC + h])
                ck.start()
                cv.start()
                copies.append(ck)
                copies.append(cv)
            for cp in copies:
                cp.wait()
            row0 = b * Sq + qb * BLK
            for h in range(H_LOC):
                qh = q_all[pl.ds(row0, BLK), h * Dh:(h + 1) * Dh]
                kh = kbuf[h].reshape(NKB * BLK, Dh)
                vh = vbuf[h].reshape(NKB * BLK, Dh)
                s = lax.dot_general(
                    qh, kh, (((1,), (1,)), ((), ())),
                    preferred_element_type=jnp.float32) * SCALE
                m = jnp.max(s, axis=1, keepdims=True)
                w = jnp.exp(s - m)
                den = jnp.sum(w, axis=1, keepdims=True)
                o = jnp.dot(w, vh, preferred_element_type=jnp.float32) / den
                ctx[pl.ds(row0, BLK), h * Dh:(h + 1) * Dh] = o
            return carry

        lax.fori_loop(0, B * QB, attn_step, 0)

        for b in range(B):
            po[b] = jnp.dot(ctx[b * Sq:(b + 1) * Sq, :], wo_ref[:, :],
                            preferred_element_type=jnp.float32)

        for st in range(N_DEV - 1):
            sc = lax.rem(j - 1 - st + 2 * N_DEV, N_DEV)
            rc = lax.rem(j - 2 - st + 2 * N_DEV, N_DEV)
            rdma = pltpu.make_async_remote_copy(
                src_ref=po.at[sc],
                dst_ref=po_rx.at[st],
                send_sem=rs_send.at[st],
                recv_sem=rs_recv.at[st],
                device_id=(right,),
                device_id_type=pl.DeviceIdType.MESH,
            )
            rdma.start()
            rdma.wait()
            po[rc] = po[rc] + po_rx[st]

        out_ref[0] = po[j]

    return pl.pallas_call(
        body,
        out_shape=jax.ShapeDtypeStruct((1, Sq, Dm), jnp.float32),
        in_specs=[
            pl.BlockSpec(memory_space=pltpu.VMEM),
            pl.BlockSpec(memory_space=pltpu.VMEM),
            pl.BlockSpec(memory_space=pltpu.ANY),
            pl.BlockSpec(memory_space=pltpu.ANY),
            pl.BlockSpec(memory_space=pltpu.VMEM),
        ],
        out_specs=pl.BlockSpec(memory_space=pltpu.VMEM),
        scratch_shapes=[
            pltpu.VMEM((B * Sq, Dm), jnp.float32),
            pltpu.VMEM((B * Sq, Dm), jnp.float32),
            pltpu.VMEM((H_LOC, NKB, BLK, Dh), jnp.float32),
            pltpu.VMEM((H_LOC, NKB, BLK, Dh), jnp.float32),
            pltpu.VMEM((B * Sq, Dm), jnp.float32),
            pltpu.VMEM((B, Sq, Dm), jnp.float32),
            pltpu.VMEM((N_DEV - 1, Sq, Dm), jnp.float32),
            pltpu.SemaphoreType.DMA((N_DEV - 1,)),
            pltpu.SemaphoreType.DMA((N_DEV - 1,)),
            pltpu.SemaphoreType.DMA((N_DEV - 1,)),
            pltpu.SemaphoreType.DMA((N_DEV - 1,)),
            pltpu.SemaphoreType.DMA((2 * H_LOC,)),
        ],
        compiler_params=pltpu.CompilerParams(collective_id=0),
    )(x, Wq, K_r, V_r, Wo)


# baseline (device time: 195630 ns/iter reference)
import jax
import jax.numpy as jnp
from jax import lax
from jax.experimental import pallas as pl
from jax.experimental.pallas import tpu as pltpu

N_DEV = 4
B, Sq, Skv, Dm = 4, 256, 4096, 1024
H_LOC, Dh = 8, 128
QB = 4
NKB = 16
BLK = 64
SCALE = 0.08838834764831843


def kernel(x, Wq, K_ext, V_ext, Wo):
    K_r = K_ext.reshape(B, NKB, QB, BLK, 32, Dh)
    V_r = V_ext.reshape(B, NKB, QB, BLK, 32, Dh)

    def body(x_ref, wq_ref, k_ref, v_ref, wo_ref, out_ref,
             xall, q_all, kbuf, vbuf, ctx, po, po_rx,
             ag_send, ag_recv, rs_send, rs_recv, dma_sems):
        j = lax.axis_index("i")
        left = lax.rem(j - 1 + N_DEV, N_DEV)
        right = lax.rem(j + 1, N_DEV)

        barrier = pltpu.get_barrier_semaphore()
        for nbr in (left, right):
            pl.semaphore_signal(barrier, inc=1, device_id=(nbr,),
                                device_id_type=pl.DeviceIdType.MESH)
        pl.semaphore_wait(barrier, 2)

        xall[pl.ds(j * Sq, Sq), :] = x_ref[0]

        for h in range(N_DEV - 1):
            c = lax.rem(j - h + 2 * N_DEV, N_DEV)
            rdma = pltpu.make_async_remote_copy(
                src_ref=xall.at[pl.ds(c * Sq, Sq), :],
                dst_ref=xall.at[pl.ds(c * Sq, Sq), :],
                send_sem=ag_send.at[h],
                recv_sem=ag_recv.at[h],
                device_id=(right,),
                device_id_type=pl.DeviceIdType.MESH,
            )
            rdma.start()
            rdma.wait()

        q_all[:, :] = jnp.dot(xall[:, :], wq_ref[:, :],
                              preferred_element_type=jnp.float32)

        def attn_step(idx, carry):
            b = idx // QB
            qb = lax.rem(idx, QB)
            copies = []
            for h in range(H_LOC):
                ha = j * H_LOC + h
                ck = pltpu.make_async_copy(
                    k_ref.at[b, :, qb, :, ha, :], kbuf.at[h],
                    dma_sems.at[h])
                cv = pltpu.make_async_copy(
                    v_ref.at[b, :, qb, :, ha, :], vbuf.at[h],
                    dma_sems.at[H_LOC + h])
                ck.start()
                cv.start()
                copies.append(ck)
                copies.append(cv)
            for cp in copies:
                cp.wait()
            row0 = b * Sq + qb * BLK
            for h in range(H_LOC):
                qh = q_all[pl.ds(row0, BLK), h * Dh:(h + 1) * Dh]
                kh = kbuf[h].reshape(NKB * BLK, Dh)
                vh = vbuf[h].reshape(NKB * BLK, Dh)
                s = lax.dot_general(
                    qh, kh, (((1,), (1,)), ((), ())),
                    preferred_element_type=jnp.float32) * SCALE
                m = jnp.max(s, axis=1, keepdims=True)
                w = jnp.exp(s - m)
                den = jnp.sum(w, axis=1, keepdims=True)
                o = jnp.dot(w, vh, preferred_element_type=jnp.float32) / den
                ctx[pl.ds(row0, BLK), h * Dh:(h + 1) * Dh] = o
            return carry

        lax.fori_loop(0, B * QB, attn_step, 0)

        for b in range(B):
            po[b] = jnp.dot(ctx[b * Sq:(b + 1) * Sq, :], wo_ref[:, :],
                            preferred_element_type=jnp.float32)

        for st in range(N_DEV - 1):
            sc = lax.rem(j - 1 - st + 2 * N_DEV, N_DEV)
            rc = lax.rem(j - 2 - st + 2 * N_DEV, N_DEV)
            rdma = pltpu.make_async_remote_copy(
                src_ref=po.at[sc],
                dst_ref=po_rx.at[st],
                send_sem=rs_send.at[st],
                recv_sem=rs_recv.at[st],
                device_id=(right,),
                device_id_type=pl.DeviceIdType.MESH,
            )
            rdma.start()
            rdma.wait()
            po[rc] = po[rc] + po_rx[st]

        out_ref[0] = po[j]

    return pl.pallas_call(
        body,
        out_shape=jax.ShapeDtypeStruct((1, Sq, Dm), jnp.float32),
        in_specs=[
            pl.BlockSpec(memory_space=pltpu.VMEM),
            pl.BlockSpec(memory_space=pltpu.VMEM),
            pl.BlockSpec(memory_space=pl.ANY),
            pl.BlockSpec(memory_space=pl.ANY),
            pl.BlockSpec(memory_space=pltpu.VMEM),
        ],
        out_specs=pl.BlockSpec(memory_space=pltpu.VMEM),
        scratch_shapes=[
            pltpu.VMEM((B * Sq, Dm), jnp.float32),
            pltpu.VMEM((B * Sq, Dm), jnp.float32),
            pltpu.VMEM((H_LOC, NKB, BLK, Dh), jnp.float32),
            pltpu.VMEM((H_LOC, NKB, BLK, Dh), jnp.float32),
            pltpu.VMEM((B * Sq, Dm), jnp.float32),
            pltpu.VMEM((B, Sq, Dm), jnp.float32),
            pltpu.VMEM((N_DEV - 1, Sq, Dm), jnp.float32),
            pltpu.SemaphoreType.DMA((N_DEV - 1,)),
            pltpu.SemaphoreType.DMA((N_DEV - 1,)),
            pltpu.SemaphoreType.DMA((N_DEV - 1,)),
            pltpu.SemaphoreType.DMA((N_DEV - 1,)),
            pltpu.SemaphoreType.DMA((2 * H_LOC,)),
        ],
        compiler_params=pltpu.CompilerParams(collective_id=0),
    )(x, Wq, K_r, V_r, Wo)
